# baseline (speedup 1.0000x reference)
import functools

import jax
import jax.numpy as jnp
from jax import lax
from jax.experimental import pallas as pl
from jax.experimental.pallas import tpu as pltpu
from jax.experimental.pallas import tpu_sc as plsc

V = 1000000
D = 64
HALF = 32
CHUNK = 256
NBUF = 4
NBLK = V // 128
VTAIL = NBLK * 128
SROWS = (V + 64) // 2


@functools.lru_cache(maxsize=None)
def _make_kernels(B):
    info = plsc.get_sparse_core_info()
    NC, NS = info.num_cores, info.num_subcores
    NW = NC * NS
    mesh = plsc.VectorSubcoreMesh(core_axis_name="c", subcore_axis_name="s")

    @functools.partial(
        pl.kernel,
        out_type=jax.ShapeDtypeStruct((SROWS, 128), jnp.float32),
        mesh=mesh,
        scratch_types=[
            pltpu.VMEM((2, 64, 128), jnp.float32),
            pltpu.VMEM((2, 64, 128), jnp.float32),
            pltpu.VMEM((32, 128), jnp.float32),
            pltpu.SemaphoreType.DMA,
            pltpu.SemaphoreType.DMA,
            pltpu.SemaphoreType.DMA,
        ],
        compiler_params=pltpu.CompilerParams(
            use_tc_tiling_on_sc=True, needs_layout_passes=False),
    )
    def transpose_k(tt_hbm, side_hbm, out_hbm, in_v, tr_v, side_v,
                    gsem, osem, ssem):
        wid = lax.axis_index("s") * NC + lax.axis_index("c")
        nb_w = (NBLK - wid + NW - 1) // NW

        def blk(k):
            return wid + k * NW

        def gstart(k, bu):
            pltpu.async_copy(
                tt_hbm.at[:, pl.ds(blk(k) * 128, 128)], in_v.at[bu], gsem)

        def gwait(bu):
            pltpu.make_async_copy(
                tt_hbm.at[:, pl.ds(0, 128)], in_v.at[bu], gsem).wait()

        def ostart(k, bu):
            pltpu.async_copy(
                tr_v.at[bu], out_hbm.at[pl.ds(blk(k) * 64, 64)], osem)

        def owait(bu):
            pltpu.make_async_copy(
                tr_v.at[bu], out_hbm.at[pl.ds(0, 64)], osem).wait()

        gstart(0, 0)
        iota = lax.iota(jnp.int32, 16)

        def step(k, carry):
            bu = k % 2

            @pl.when(k + 1 < nb_w)
            def _():
                gstart(k + 1, (k + 1) % 2)

            gwait(bu)

            @pl.when(k >= 2)
            def _():
                owait(bu)

            for r in range(64):
                for g in range(8):
                    li = 2 * r + (1 if g >= 4 else 0)
                    cols = iota + (16 * (g % 4))
                    vals = plsc.load_gather(
                        in_v.at[bu], [cols, jnp.full((16,), li, jnp.int32)])
                    tr_v[bu, r, pl.ds(16 * g, 16)] = vals
            ostart(k, bu)
            return carry

        lax.fori_loop(0, nb_w, step, 0)
        owait(0)
        owait(1)

        @pl.when(wid == 0)
        def _():
            pltpu.sync_copy(side_hbm, side_v)
            pltpu.async_copy(
                side_v, out_hbm.at[pl.ds(VTAIL // 2, 32)], ssem).wait()

    b_per_w = B // NW
    assert b_per_w * NW == B and b_per_w % CHUNK == 0
    n_chunks = b_per_w // CHUNK
    assert n_chunks > NBUF
    C2 = 2 * CHUNK

    @functools.partial(
        pl.kernel,
        out_type=jax.ShapeDtypeStruct((NW, n_chunks, C2, HALF), jnp.float32),
        mesh=mesh,
        scratch_types=[
            pltpu.VMEM((n_chunks, C2), jnp.int32),
            pltpu.VMEM((NBUF, C2, HALF), jnp.float32),
            pltpu.SemaphoreType.DMA,
            pltpu.SemaphoreType.DMA,
        ],
        compiler_params=pltpu.CompilerParams(use_tc_tiling_on_sc=False),
    )
    def gather_k(table_hbm, idx_hbm, out_hbm, idx_v, rows_v, gsem, osem):
        wid = lax.axis_index("s") * NC + lax.axis_index("c")
        pltpu.sync_copy(idx_hbm.at[wid], idx_v)

        def gstart(j, b):
            pltpu.async_copy(table_hbm.at[idx_v.at[j]], rows_v.at[b], gsem)

        def gwait(b):
            pltpu.make_async_copy(
                table_hbm.at[pl.ds(0, C2)], rows_v.at[b], gsem).wait()

        def ostart(j, b):
            pltpu.async_copy(rows_v.at[b], out_hbm.at[wid, j], osem)

        def owait(b):
            pltpu.make_async_copy(
                rows_v.at[b], out_hbm.at[wid, 0], osem).wait()

        for b in range(NBUF):
            gstart(b, b)

        def step(j, carry):
            b = j % NBUF
            gwait(b)
            ostart(j, b)
            owait(b)
            gstart(j + NBUF, b)
            return carry

        lax.fori_loop(0, n_chunks - NBUF, step, 0)

        for j in range(n_chunks - NBUF, n_chunks):
            b = j % NBUF
            gwait(b)
            ostart(j, b)
        for j in range(n_chunks - NBUF, n_chunks):
            owait(j % NBUF)

    return transpose_k, gather_k, NW, n_chunks, C2


def kernel(x, latent_tdirs):
    n, t = x.shape[0], x.shape[1]
    B = n * t
    transpose_k, gather_k, NW, n_chunks, C2 = _make_kernels(B)

    tt = latent_tdirs.T
    side = latent_tdirs[VTAIL:, :].reshape(32, 128)
    scratch = transpose_k(tt, side)
    table32 = scratch.reshape(4 * SROWS, HALF)

    idx = x.reshape(-1)
    idx2 = (2 * idx[:, None] + jnp.arange(2, dtype=jnp.int32)[None, :])
    idx2 = idx2.reshape(NW, n_chunks, C2)
    out = gather_k(table32, idx2)
    return out.reshape(n, t, D)


# speedup vs baseline: 1.1553x; 1.1553x over previous
import functools

import jax
import jax.numpy as jnp
from jax import lax
from jax.experimental import pallas as pl
from jax.experimental.pallas import tpu as pltpu
from jax.experimental.pallas import tpu_sc as plsc

V = 1000000
D = 64
HALF = 32
CHUNK = 256
NBUF = 4
TW = 512
NGRID = (V + TW - 1) // TW
SROWS = NGRID * TW // 2


@functools.lru_cache(maxsize=None)
def _make_kernels(B):
    info = plsc.get_sparse_core_info()
    NC, NS = info.num_cores, info.num_subcores
    NW = NC * NS
    mesh = plsc.VectorSubcoreMesh(core_axis_name="c", subcore_axis_name="s")

    def transpose_body(tt_ref, out_ref):
        x = tt_ref[...]
        xt = jnp.transpose(x, (1, 0))
        xr = xt.reshape(TW // 2, 2, 64)
        out_ref[:, 0:64] = xr[:, 0, :]
        out_ref[:, 64:128] = xr[:, 1, :]

    transpose_k = pl.pallas_call(
        transpose_body,
        grid=(NGRID,),
        in_specs=[pl.BlockSpec((64, TW), lambda j: (0, j))],
        out_specs=pl.BlockSpec((TW // 2, 128), lambda j: (j, 0)),
        out_shape=jax.ShapeDtypeStruct((SROWS, 128), jnp.float32),
    )

    b_per_w = B // NW
    assert b_per_w * NW == B and b_per_w % CHUNK == 0
    n_chunks = b_per_w // CHUNK
    assert n_chunks > NBUF
    C2 = 2 * CHUNK

    @functools.partial(
        pl.kernel,
        out_type=jax.ShapeDtypeStruct((NW, n_chunks, C2, HALF), jnp.float32),
        mesh=mesh,
        scratch_types=[
            pltpu.VMEM((n_chunks, C2), jnp.int32),
            pltpu.VMEM((NBUF, C2, HALF), jnp.float32),
            pltpu.SemaphoreType.DMA,
            pltpu.SemaphoreType.DMA,
        ],
        compiler_params=pltpu.CompilerParams(use_tc_tiling_on_sc=False),
    )
    def gather_k(table_hbm, idx_hbm, out_hbm, idx_v, rows_v, gsem, osem):
        wid = lax.axis_index("s") * NC + lax.axis_index("c")
        pltpu.sync_copy(idx_hbm.at[wid], idx_v)

        def gstart(j, b):
            pltpu.async_copy(table_hbm.at[idx_v.at[j]], rows_v.at[b], gsem)

        def gwait(b):
            pltpu.make_async_copy(
                table_hbm.at[pl.ds(0, C2)], rows_v.at[b], gsem).wait()

        def ostart(j, b):
            pltpu.async_copy(rows_v.at[b], out_hbm.at[wid, j], osem)

        def owait(b):
            pltpu.make_async_copy(
                rows_v.at[b], out_hbm.at[wid, 0], osem).wait()

        for b in range(NBUF):
            gstart(b, b)

        def step(j, carry):
            b = j % NBUF
            gwait(b)
            ostart(j, b)
            owait(b)
            gstart(j + NBUF, b)
            return carry

        lax.fori_loop(0, n_chunks - NBUF, step, 0)

        for j in range(n_chunks - NBUF, n_chunks):
            b = j % NBUF
            gwait(b)
            ostart(j, b)
        for j in range(n_chunks - NBUF, n_chunks):
            owait(j % NBUF)

    return transpose_k, gather_k, NW, n_chunks, C2


def kernel(x, latent_tdirs):
    n, t = x.shape[0], x.shape[1]
    B = n * t
    transpose_k, gather_k, NW, n_chunks, C2 = _make_kernels(B)

    tt = latent_tdirs.T
    scratch = transpose_k(tt)
    table32 = scratch.reshape(4 * SROWS, HALF)

    idx = x.reshape(-1)
    idx2 = (2 * idx[:, None] + jnp.arange(2, dtype=jnp.int32)[None, :])
    idx2 = idx2.reshape(NW, n_chunks, C2)
    out = gather_k(table32, idx2)
    return out.reshape(n, t, D)
